# Initial kernel scaffold; baseline (speedup 1.0000x reference)
import jax
import jax.numpy as jnp
from jax.experimental import pallas as pl


def kernel(z, W, thresholds):
    raise NotImplementedError("write your pallas kernel here")


# speedup vs baseline: 1667.8603x; 1667.8603x over previous
import functools

import jax
import jax.numpy as jnp
from jax.experimental import pallas as pl

_ROWS_PER_BLOCK = 256


def _fused_body(z_ref, w_ref, t_ref, o_ref, *, k: int):
    f = jax.lax.dot_general(
        z_ref[...],
        w_ref[...],
        (((1,), (1,)), ((), ())),
        preferred_element_type=jnp.float32,
    )
    f = jnp.maximum(f - t_ref[...], 0.0)

    if k <= 0:
        o_ref[...] = jax.nn.sigmoid(f)
        return

    n_pos = jnp.sum((f > 0.0).astype(jnp.int32), axis=1, keepdims=True)
    need_search = jnp.any(n_pos > k)

    @pl.when(jnp.logical_not(need_search))
    def _common():
        o_ref[...] = jax.nn.sigmoid(f)

    @pl.when(need_search)
    def _rare():
        bits = jax.lax.bitcast_convert_type(f, jnp.int32)

        def step(_, carry):
            lo, hi = carry
            mid = lo + ((hi - lo + 1) >> 1)
            cnt = jnp.sum((bits >= mid).astype(jnp.int32), axis=1,
                          keepdims=True)
            pred = cnt >= k
            lo = jnp.where(pred, mid, lo)
            hi = jnp.where(pred, hi, mid - 1)
            return lo, hi

        rows, cols = f.shape
        lo0 = jnp.zeros((rows, 1), jnp.int32)
        hi0 = jnp.full((rows, 1), jnp.int32(0x7F7FFFFF))
        lo, _ = jax.lax.fori_loop(0, 31, step, (lo0, hi0))

        gt = bits > lo
        ties = bits == lo
        m = k - jnp.sum(gt.astype(jnp.int32), axis=1, keepdims=True)
        col = jax.lax.broadcasted_iota(jnp.int32, f.shape, 1)

        def col_step(_, carry):
            clo, chi = carry
            cmid = clo + ((chi - clo + 1) >> 1)
            cnt = jnp.sum((ties & (col < cmid)).astype(jnp.int32), axis=1,
                          keepdims=True)
            pred = cnt <= m
            clo = jnp.where(pred, cmid, clo)
            chi = jnp.where(pred, chi, cmid - 1)
            return clo, chi

        nbits = max(1, (cols + 1).bit_length())
        clo0 = jnp.zeros((rows, 1), jnp.int32)
        chi0 = jnp.full((rows, 1), jnp.int32(cols))
        clo, _ = jax.lax.fori_loop(0, nbits, col_step, (clo0, chi0))
        mask = gt | (ties & (col < clo))
        o_ref[...] = jax.nn.sigmoid(jnp.where(mask, f, 0.0))


@jax.jit
def kernel(z, W, thresholds):
    n_rows, d = z.shape
    n_factors = W.shape[0]
    target_sparsity = 0.3
    k = int(n_factors * (1.0 - target_sparsity))

    th2d = thresholds.reshape(1, n_factors).astype(jnp.float32)
    rb = min(_ROWS_PER_BLOCK, n_rows)
    grid = (n_rows // rb,)

    out = pl.pallas_call(
        functools.partial(_fused_body, k=k),
        grid=grid,
        in_specs=[
            pl.BlockSpec((rb, d), lambda i: (i, 0)),
            pl.BlockSpec((n_factors, d), lambda i: (0, 0)),
            pl.BlockSpec((1, n_factors), lambda i: (0, 0)),
        ],
        out_specs=pl.BlockSpec((rb, n_factors), lambda i: (i, 0)),
        out_shape=jax.ShapeDtypeStruct((n_rows, n_factors), jnp.float32),
    )(z, W, th2d)
    return out
